# baseline (speedup 1.0000x reference)
import functools

import jax
import jax.numpy as jnp
from jax.experimental import pallas as pl

B = 16
T0 = 128
CIN = 263
WIDTH = 512
CODE_DIM = 512
NB_CODE = 1024
DOWN_T = 3
DEPTH = 3
DGR = 3
N_DEM_LAYERS = 8

_F32 = jnp.float32


def _mm(a, b):
    return jax.lax.dot_general(
        a, b, (((1,), (0,)), ((), ())), preferred_element_type=_F32)


def _mm_t(a, b):
    return jax.lax.dot_general(
        a, b, (((1,), (1,)), ((), ())), preferred_element_type=_F32)


def _conv_kernel(x_ref, w_ref, b_ref, o_ref, *, dil, relu):
    K = w_ref.shape[0]
    Tout = o_ref.shape[1]
    Ci = x_ref.shape[2]
    O = o_ref.shape[2]
    x = x_ref[...]
    acc = _mm(x[:, 0:Tout, :].reshape(-1, Ci), w_ref[0])
    for k in range(1, K):
        acc = acc + _mm(
            x[:, k * dil:k * dil + Tout, :].reshape(-1, Ci), w_ref[k])
    acc = acc + b_ref[...]
    if relu:
        acc = jnp.maximum(acc, 0.0)
    o_ref[...] = acc.reshape(B, Tout, O)


def _res_kernel(h_ref, w3_ref, b3_ref, w1_ref, b1_ref, o_ref, *, dil):
    h = h_ref[...]
    Tt = h.shape[1]
    C = h.shape[2]
    r = jnp.maximum(h, 0.0)
    z = jnp.zeros((B, dil, C), _F32)
    rp = jnp.concatenate([z, r, z], axis=1)
    acc = _mm(rp[:, 0:Tt, :].reshape(-1, C), w3_ref[0])
    for k in range(1, 3):
        acc = acc + _mm(
            rp[:, k * dil:k * dil + Tt, :].reshape(-1, C), w3_ref[k])
    acc = acc + b3_ref[...]
    r2 = jnp.maximum(acc, 0.0)
    r3 = _mm(r2, w1_ref[...]) + b1_ref[...]
    o_ref[...] = h + r3.reshape(B, Tt, C)


def _quant_kernel(x_ref, cb_ref, dem_ref, hq_ref, loss_ref, perp_ref):
    x = x_ref[...]
    M = x.shape[0]
    C = x.shape[1]
    NB = cb_ref.shape[0]
    m = jnp.mean(x, axis=1, keepdims=True)
    xc = x - m
    v = jnp.mean(xc * xc, axis=1, keepdims=True)
    zf = xc * jax.lax.rsqrt(v + 1e-5)
    cb = cb_ref[...]
    cb_n = jnp.sum(cb * cb, axis=1)[None, :]
    zf_n = jnp.sum(zf * zf, axis=1, keepdims=True)
    d2 = zf_n + cb_n - 2.0 * _mm_t(zf, cb)
    mn = jnp.min(d2, axis=1, keepdims=True)
    iota = jax.lax.broadcasted_iota(jnp.int32, (M, NB), 1)
    idx = jnp.min(jnp.where(d2 <= mn, iota, NB), axis=1, keepdims=True)
    onehot = (iota == idx).astype(_F32)
    zq = _mm(onehot, cb)
    diff = zq - zf
    loss_ref[...] = 2.0 * jnp.mean(diff * diff, keepdims=True)
    em = jnp.mean(onehot, axis=0, keepdims=True)
    ent = jnp.sum(em * jnp.log(em + 1e-10), axis=1, keepdims=True)
    perp_ref[...] = jnp.exp(-ent)
    h = zq
    for i in range(N_DEM_LAYERS):
        h = h + jnp.maximum(_mm_t(h, dem_ref[i]), 0.0)
    hq_ref[...] = h


def _vmem_specs(n):
    return [pl.BlockSpec(memory_space=pl.ANY)] * n


def _call_conv(x_pad, w, b, Tout, dil, relu):
    O = w.shape[2]
    return pl.pallas_call(
        functools.partial(_conv_kernel, dil=dil, relu=relu),
        out_shape=jax.ShapeDtypeStruct((B, Tout, O), _F32),
    )(x_pad, w, b)


def _call_res(h, w3, b3, w1, b1, dil):
    return pl.pallas_call(
        functools.partial(_res_kernel, dil=dil),
        out_shape=jax.ShapeDtypeStruct(h.shape, _F32),
    )(h, w3, b3, w1, b1)


def _prep_conv_w(w):
    return jnp.transpose(w, (2, 1, 0))


def _pad_t(x, pad):
    return jnp.pad(x, ((0, 0), (pad, pad), (0, 0)))


def _conv3(x, w, b, dil, relu=False):
    Tt = x.shape[1]
    return _call_conv(_pad_t(x, dil), _prep_conv_w(w), b[None, :], Tt, dil,
                      relu)


def _down4(x, w, b):
    Tt = x.shape[1]
    To = Tt // 2
    C = x.shape[2]
    xp = jnp.pad(x, ((0, 0), (1, 3), (0, 0)))
    u = xp.reshape(B, To + 2, 2 * C)
    wt = _prep_conv_w(w)
    O = wt.shape[2]
    wp = jnp.stack(
        [wt[0:2].reshape(2 * C, O), wt[2:4].reshape(2 * C, O)], axis=0)
    return _call_conv(u, wp, b[None, :], To, 1, False)


def kernel(x, enc_params, dec_params, dem_params, codebook):
    it = iter(enc_params)
    w, b = next(it)
    h = _conv3(x, w, b, 1, relu=True)
    for i in range(DOWN_T):
        w, b = next(it)
        h = _down4(h, w, b)
        for j in range(DEPTH):
            d = DGR ** j
            w1, b1 = next(it)
            w2, b2 = next(it)
            h = _call_res(h, _prep_conv_w(w1), b1[None, :],
                          jnp.transpose(w2[:, :, 0]), b2[None, :], d)
    w, b = next(it)
    x_enc = _conv3(h, w, b, 1, relu=False)
    N = x_enc.shape[1]

    dem_w = jnp.stack(dem_params)
    hq_t, loss, perp = pl.pallas_call(
        _quant_kernel,
        out_shape=(
            jax.ShapeDtypeStruct((B * N, CODE_DIM), _F32),
            jax.ShapeDtypeStruct((1, 1), _F32),
            jax.ShapeDtypeStruct((1, 1), _F32),
        ),
    )(x_enc.reshape(B * N, CODE_DIM), codebook, dem_w)
    h = hq_t.reshape(B, N, CODE_DIM)

    it = iter(dec_params)
    w, b = next(it)
    h = _conv3(h, w, b, 1, relu=True)
    for i in range(DOWN_T):
        for j in range(DEPTH):
            d = DGR ** j
            w1, b1 = next(it)
            w2, b2 = next(it)
            h = _call_res(h, _prep_conv_w(w1), b1[None, :],
                          jnp.transpose(w2[:, :, 0]), b2[None, :], d)
        h = jnp.repeat(h, 2, axis=1)
        w, b = next(it)
        h = _conv3(h, w, b, 1, relu=False)
    w, b = next(it)
    h = _conv3(h, w, b, 1, relu=True)
    w, b = next(it)
    out_t = _conv3(h, w, b, 1, relu=False)
    out = jnp.transpose(out_t, (0, 2, 1))
    return out, loss[0, 0], perp[0, 0]


# speedup vs baseline: 1.1467x; 1.1467x over previous
import functools

import jax
import jax.numpy as jnp
from jax.experimental import pallas as pl

B = 16
CIN = 263
WIDTH = 512
CODE_DIM = 512
NB_CODE = 1024
N_DEM_LAYERS = 8
DILS = (1, 3, 9)

_F32 = jnp.float32
_BF16 = jnp.bfloat16


def _mm(a, b):
    return jax.lax.dot_general(
        a.astype(_BF16), b.astype(_BF16), (((1,), (0,)), ((), ())),
        preferred_element_type=_F32)


def _mm_t(a, b):
    return jax.lax.dot_general(
        a.astype(_BF16), b.astype(_BF16), (((1,), (1,)), ((), ())),
        preferred_element_type=_F32)


def _mm_exact(a, b):
    return jax.lax.dot_general(
        a, b, (((1,), (0,)), ((), ())), preferred_element_type=_F32,
        precision=jax.lax.Precision.HIGHEST)


def _pad_t(h, lo, hi):
    zlo = jnp.zeros((h.shape[0], lo, h.shape[2]), _F32)
    zhi = jnp.zeros((h.shape[0], hi, h.shape[2]), _F32)
    return jnp.concatenate([zlo, h, zhi], axis=1)


def _conv3(h, w_ref, b_ref, dil, relu):
    Tt = h.shape[1]
    Ci = h.shape[2]
    rp = _pad_t(h, dil, dil)
    p = jnp.concatenate(
        [rp[:, k * dil:k * dil + Tt, :] for k in range(3)], axis=2)
    acc = _mm(p.reshape(-1, 3 * Ci), w_ref[...]) + b_ref[...]
    if relu:
        acc = jnp.maximum(acc, 0.0)
    return acc.reshape(B, Tt, w_ref.shape[-1])


def _resnets(h, wr3_ref, br3_ref, wr1_ref, br1_ref):
    C = h.shape[2]
    Tt = h.shape[1]
    for j, d in enumerate(DILS):
        r = jnp.maximum(h, 0.0)
        rp = _pad_t(r, d, d)
        p = jnp.concatenate(
            [rp[:, k * d:k * d + Tt, :] for k in range(3)], axis=2)
        acc = _mm(p.reshape(-1, 3 * C), wr3_ref[j]) + br3_ref[j]
        r2 = jnp.maximum(acc, 0.0)
        r3 = _mm(r2, wr1_ref[j]) + br1_ref[j]
        h = h + r3.reshape(B, Tt, C)
    return h


def _enc_stage_kernel(h_ref, wd_ref, bd_ref, wr3_ref, br3_ref, wr1_ref,
                      br1_ref, o_ref):
    h = h_ref[...]
    Tt = h.shape[1]
    C = h.shape[2]
    To = Tt // 2
    xp = _pad_t(h, 1, 3)
    xr = xp.reshape(B, (Tt + 4) // 2, 2, C)
    xe = xr[:, :, 0:1, :].reshape(B, (Tt + 4) // 2, C)
    xo = xr[:, :, 1:2, :].reshape(B, (Tt + 4) // 2, C)
    p = jnp.concatenate(
        [xe[:, 0:To, :], xo[:, 0:To, :], xe[:, 1:To + 1, :],
         xo[:, 1:To + 1, :]], axis=2)
    y = _mm(p.reshape(-1, 4 * C), wd_ref[...]) + bd_ref[...]
    h = y.reshape(B, To, wd_ref.shape[-1])
    o_ref[...] = _resnets(h, wr3_ref, br3_ref, wr1_ref, br1_ref)


def _dec_stage_kernel(h_ref, wr3_ref, br3_ref, wr1_ref, br1_ref, wu_ref,
                      bu_ref, o_ref):
    h = _resnets(h_ref[...], wr3_ref, br3_ref, wr1_ref, br1_ref)
    To = h.shape[1]
    C = h.shape[2]
    u = jnp.broadcast_to(h[:, :, None, :], (B, To, 2, C)).reshape(B, 2 * To, C)
    o_ref[...] = _conv3(u, wu_ref, bu_ref, 1, False)


def _mid_kernel(h_ref, we_ref, be_ref, cb_ref, dem_ref, wd_ref, bd_ref,
                o_ref, loss_ref, perp_ref):
    xe = _conv3(h_ref[...], we_ref, be_ref, 1, False)
    N = xe.shape[1]
    C = xe.shape[2]
    x = xe.reshape(B * N, C)
    M = B * N
    NB = cb_ref.shape[0]
    m = jnp.mean(x, axis=1, keepdims=True)
    xc = x - m
    v = jnp.mean(xc * xc, axis=1, keepdims=True)
    zf = xc / jnp.sqrt(v + 1e-5)
    cb = cb_ref[...]
    cb_n = jnp.sum(cb * cb, axis=1)[None, :]
    zf_n = jnp.sum(zf * zf, axis=1, keepdims=True)
    d2 = zf_n + cb_n - 2.0 * _mm_t(zf, cb)
    mn = jnp.min(d2, axis=1, keepdims=True)
    iota = jax.lax.broadcasted_iota(jnp.int32, (M, NB), 1)
    idx = jnp.min(jnp.where(d2 <= mn, iota, NB), axis=1, keepdims=True)
    onehot = (iota == idx).astype(_F32)
    zq = _mm_exact(onehot, cb)
    diff = zq - zf
    loss_ref[...] = 2.0 * jnp.mean(diff * diff, keepdims=True)
    em = jnp.mean(onehot, axis=0, keepdims=True)
    ent = jnp.sum(em * jnp.log(em + 1e-10), axis=1, keepdims=True)
    perp_ref[...] = jnp.exp(-ent)
    h = zf + (zq - zf)
    for i in range(N_DEM_LAYERS):
        h = h + jnp.maximum(_mm_t(h, dem_ref[i]), 0.0)
    o_ref[...] = _conv3(h.reshape(B, N, C), wd_ref, bd_ref, 1, True)


def _dec_out_kernel(h_ref, w1_ref, b1_ref, w2_ref, b2_ref, o_ref):
    h = _conv3(h_ref[...], w1_ref, b1_ref, 1, True)
    o_ref[...] = _conv3(h, w2_ref, b2_ref, 1, False)


def _kmaj(w):
    return jnp.transpose(w, (2, 1, 0)).reshape(-1, w.shape[0])


def _call(fn, args, out_shape):
    return pl.pallas_call(fn, out_shape=out_shape)(*args)


def _stage_weights(params):
    wr3 = jnp.stack([_kmaj(w) for (w, _, _, _) in params], axis=0)
    br3 = jnp.stack([b[None, :] for (_, b, _, _) in params], axis=0)
    wr1 = jnp.stack([jnp.transpose(w[:, :, 0]) for (_, _, w, _) in params],
                    axis=0)
    br1 = jnp.stack([b[None, :] for (_, _, _, b) in params], axis=0)
    return wr3, br3, wr1, br1


def kernel(x, enc_params, dec_params, dem_params, codebook):
    f32 = jax.ShapeDtypeStruct
    it = iter(enc_params)
    w, b = next(it)
    h0 = jax.lax.conv_general_dilated(
        jnp.transpose(x, (0, 2, 1)), w, window_strides=(1,),
        padding=[(1, 1)], dimension_numbers=('NCH', 'OIH', 'NCH'))
    h = jnp.transpose(jax.nn.relu(h0 + b[None, :, None]), (0, 2, 1))
    for i in range(3):
        wd, bd = next(it)
        res = []
        for j in range(3):
            w1, b1 = next(it)
            w2, b2 = next(it)
            res.append((w1, b1, w2, b2))
        wr3, br3, wr1, br1 = _stage_weights(res)
        To = h.shape[1] // 2
        h = _call(_enc_stage_kernel,
                  (h, _kmaj(wd), bd[None, :], wr3, br3, wr1, br1),
                  f32((B, To, WIDTH), _F32))
    we, be = next(it)

    dit = iter(dec_params)
    wdi, bdi = next(dit)
    dem_w = jnp.stack(dem_params)
    N = h.shape[1]
    h, loss, perp = _call(
        _mid_kernel,
        (h, _kmaj(we), be[None, :], codebook, dem_w,
         _kmaj(wdi), bdi[None, :]),
        (f32((B, N, CODE_DIM), _F32), f32((1, 1), _F32), f32((1, 1), _F32)))

    for i in range(3):
        res = []
        for j in range(3):
            w1, b1 = next(dit)
            w2, b2 = next(dit)
            res.append((w1, b1, w2, b2))
        wr3, br3, wr1, br1 = _stage_weights(res)
        wu, bu = next(dit)
        To = h.shape[1] * 2
        h = _call(_dec_stage_kernel,
                  (h, wr3, br3, wr1, br1, _kmaj(wu), bu[None, :]),
                  f32((B, To, WIDTH), _F32))
    w1, b1 = next(dit)
    w2, b2 = next(dit)
    out_t = _call(_dec_out_kernel,
                  (h, _kmaj(w1), b1[None, :], _kmaj(w2), b2[None, :]),
                  f32((B, h.shape[1], CIN), _F32))
    out = jnp.transpose(out_t, (0, 2, 1))
    return out, loss[0, 0], perp[0, 0]


# speedup vs baseline: 1.4255x; 1.2431x over previous
import functools

import jax
import jax.numpy as jnp
from jax.experimental import pallas as pl

B = 16
CIN = 263
WIDTH = 512
CODE_DIM = 512
NB_CODE = 1024
N_DEM_LAYERS = 8
DILS = (1, 3, 9)

_F32 = jnp.float32
_BF16 = jnp.bfloat16


def _mm(a, b):
    return jax.lax.dot_general(
        a.astype(_BF16), b.astype(_BF16), (((1,), (0,)), ((), ())),
        preferred_element_type=_F32)


def _mm_t(a, b):
    return jax.lax.dot_general(
        a.astype(_BF16), b.astype(_BF16), (((1,), (1,)), ((), ())),
        preferred_element_type=_F32)


def _mm_exact(a, b):
    return jax.lax.dot_general(
        a, b, (((1,), (0,)), ((), ())), preferred_element_type=_F32,
        precision=jax.lax.Precision.HIGHEST)


def _pad_t(h, lo, hi):
    zlo = jnp.zeros((h.shape[0], lo, h.shape[2]), _F32)
    zhi = jnp.zeros((h.shape[0], hi, h.shape[2]), _F32)
    return jnp.concatenate([zlo, h, zhi], axis=1)


def _conv3(h, w_ref, b_ref, dil, relu):
    Tt = h.shape[1]
    Ci = h.shape[2]
    rp = _pad_t(h, dil, dil)
    p = jnp.concatenate(
        [rp[:, k * dil:k * dil + Tt, :] for k in range(3)], axis=2)
    acc = _mm(p.reshape(-1, 3 * Ci), w_ref[...]) + b_ref[...]
    if relu:
        acc = jnp.maximum(acc, 0.0)
    return acc.reshape(B, Tt, w_ref.shape[-1])


def _resnets(h, wr3_ref, br3_ref, wr1_ref, br1_ref):
    C = h.shape[2]
    Tt = h.shape[1]
    for j, d in enumerate(DILS):
        r = jnp.maximum(h, 0.0)
        rp = _pad_t(r, d, d)
        p = jnp.concatenate(
            [rp[:, k * d:k * d + Tt, :] for k in range(3)], axis=2)
        acc = _mm(p.reshape(-1, 3 * C), wr3_ref[j]) + br3_ref[j]
        r2 = jnp.maximum(acc, 0.0)
        r3 = _mm(r2, wr1_ref[j]) + br1_ref[j]
        h = h + r3.reshape(B, Tt, C)
    return h


def _enc_stage_kernel(h_ref, wd_ref, bd_ref, wr3_ref, br3_ref, wr1_ref,
                      br1_ref, o_ref):
    h = h_ref[...]
    Tt = h.shape[1]
    C = h.shape[2]
    To = Tt // 2
    xp = _pad_t(h, 1, 3)
    xr = xp.reshape(B, (Tt + 4) // 2, 2, C)
    xe = xr[:, :, 0:1, :].reshape(B, (Tt + 4) // 2, C)
    xo = xr[:, :, 1:2, :].reshape(B, (Tt + 4) // 2, C)
    p = jnp.concatenate(
        [xe[:, 0:To, :], xo[:, 0:To, :], xe[:, 1:To + 1, :],
         xo[:, 1:To + 1, :]], axis=2)
    y = _mm(p.reshape(-1, 4 * C), wd_ref[...]) + bd_ref[...]
    h = y.reshape(B, To, wd_ref.shape[-1])
    o_ref[...] = _resnets(h, wr3_ref, br3_ref, wr1_ref, br1_ref)


def _dec_stage_kernel(h_ref, wr3_ref, br3_ref, wr1_ref, br1_ref, wu_ref,
                      bu_ref, o_ref):
    h = _resnets(h_ref[...], wr3_ref, br3_ref, wr1_ref, br1_ref)
    To = h.shape[1]
    C = h.shape[2]
    u = jnp.broadcast_to(h[:, :, None, :], (B, To, 2, C)).reshape(B, 2 * To, C)
    o_ref[...] = _conv3(u, wu_ref, bu_ref, 1, False)


def _mid_kernel(h_ref, we_ref, be_ref, cb_ref, dem_ref, wd_ref, bd_ref,
                o_ref, loss_ref, perp_ref):
    xe = _conv3(h_ref[...], we_ref, be_ref, 1, False)
    N = xe.shape[1]
    C = xe.shape[2]
    x = xe.reshape(B * N, C)
    M = B * N
    NB = cb_ref.shape[0]
    m = jnp.mean(x, axis=1, keepdims=True)
    xc = x - m
    v = jnp.mean(xc * xc, axis=1, keepdims=True)
    zf = xc / jnp.sqrt(v + 1e-5)
    cb = cb_ref[...]
    cb_n = jnp.sum(cb * cb, axis=1)[None, :]
    zf_n = jnp.sum(zf * zf, axis=1, keepdims=True)
    d2 = zf_n + cb_n - 2.0 * _mm_t(zf, cb)
    mn = jnp.min(d2, axis=1, keepdims=True)
    iota = jax.lax.broadcasted_iota(jnp.int32, (M, NB), 1)
    idx = jnp.min(jnp.where(d2 <= mn, iota, NB), axis=1, keepdims=True)
    onehot = (iota == idx).astype(_F32)
    zq = _mm_exact(onehot, cb)
    diff = zq - zf
    loss_ref[...] = 2.0 * jnp.mean(diff * diff, keepdims=True)
    em = jnp.mean(onehot, axis=0, keepdims=True)
    ent = jnp.sum(em * jnp.log(em + 1e-10), axis=1, keepdims=True)
    perp_ref[...] = jnp.exp(-ent)
    h = zf + (zq - zf)
    for i in range(N_DEM_LAYERS):
        h = h + jnp.maximum(_mm_t(h, dem_ref[i]), 0.0)
    o_ref[...] = _conv3(h.reshape(B, N, C), wd_ref, bd_ref, 1, True)


def _dec_out_kernel(h_ref, w1_ref, b1_ref, w2_ref, b2_ref, o_ref):
    h = _conv3(h_ref[...], w1_ref, b1_ref, 1, True)
    o_ref[...] = _conv3(h, w2_ref, b2_ref, 1, False)


def _kmaj(w):
    return jnp.transpose(w, (2, 1, 0)).reshape(-1, w.shape[0]).astype(_BF16)


def _call(fn, args, out_shape):
    return pl.pallas_call(fn, out_shape=out_shape)(*args)


def _stage_weights(params):
    wr3 = jnp.stack([_kmaj(w) for (w, _, _, _) in params], axis=0)
    br3 = jnp.stack([b[None, :] for (_, b, _, _) in params], axis=0)
    wr1 = jnp.stack([jnp.transpose(w[:, :, 0]) for (_, _, w, _) in params],
                    axis=0).astype(_BF16)
    br1 = jnp.stack([b[None, :] for (_, _, _, b) in params], axis=0)
    return wr3, br3, wr1, br1


def kernel(x, enc_params, dec_params, dem_params, codebook):
    f32 = jax.ShapeDtypeStruct
    it = iter(enc_params)
    w, b = next(it)
    h0 = jax.lax.conv_general_dilated(
        jnp.transpose(x, (0, 2, 1)), w, window_strides=(1,),
        padding=[(1, 1)], dimension_numbers=('NCH', 'OIH', 'NCH'))
    h = jnp.transpose(jax.nn.relu(h0 + b[None, :, None]), (0, 2, 1))
    for i in range(3):
        wd, bd = next(it)
        res = []
        for j in range(3):
            w1, b1 = next(it)
            w2, b2 = next(it)
            res.append((w1, b1, w2, b2))
        wr3, br3, wr1, br1 = _stage_weights(res)
        To = h.shape[1] // 2
        h = _call(_enc_stage_kernel,
                  (h, _kmaj(wd), bd[None, :], wr3, br3, wr1, br1),
                  f32((B, To, WIDTH), _F32))
    we, be = next(it)

    dit = iter(dec_params)
    wdi, bdi = next(dit)
    dem_w = jnp.stack(dem_params).astype(_BF16)
    N = h.shape[1]
    h, loss, perp = _call(
        _mid_kernel,
        (h, _kmaj(we), be[None, :], codebook, dem_w,
         _kmaj(wdi), bdi[None, :]),
        (f32((B, N, CODE_DIM), _F32), f32((1, 1), _F32), f32((1, 1), _F32)))

    for i in range(3):
        res = []
        for j in range(3):
            w1, b1 = next(dit)
            w2, b2 = next(dit)
            res.append((w1, b1, w2, b2))
        wr3, br3, wr1, br1 = _stage_weights(res)
        wu, bu = next(dit)
        To = h.shape[1] * 2
        h = _call(_dec_stage_kernel,
                  (h, wr3, br3, wr1, br1, _kmaj(wu), bu[None, :]),
                  f32((B, To, WIDTH), _F32))
    w1, b1 = next(dit)
    w2, b2 = next(dit)
    out_t = _call(_dec_out_kernel,
                  (h, _kmaj(w1), b1[None, :], _kmaj(w2), b2[None, :]),
                  f32((B, h.shape[1], CIN), _F32))
    out = jnp.transpose(out_t, (0, 2, 1))
    return out, loss[0, 0], perp[0, 0]
